# Initial kernel scaffold; baseline (speedup 1.0000x reference)
import jax
import jax.numpy as jnp
from jax.experimental import pallas as pl


def kernel(x, edge_index, edge_label_index, embed, W1, b1, W2, b2, fc_w, fc_b):
    raise NotImplementedError("write your pallas kernel here")


# speedup vs baseline: 24.6261x; 24.6261x over previous
import functools

import jax
import jax.numpy as jnp
from jax import lax
from jax.experimental import pallas as pl
from jax.experimental.pallas import tpu as pltpu
from jax.experimental.pallas import tpu_sc as plsc

N = 100000
E = 3200000
F = 16
LANES = 128
R = E // LANES
NC = 2
NS = 16
RPC = R // NC
RPT = RPC // NS
RREM = RPC - RPT * NS
NPT = N // NS
ZROWS = 1250
NB = 20000
GRID = N // NB

_MESH = functools.partial(
    plsc.VectorSubcoreMesh, core_axis_name="c", subcore_axis_name="s",
    num_cores=NC, num_subcores=NS)
_SC_PARAMS = pltpu.CompilerParams(use_tc_tiling_on_sc=False,
                                  needs_layout_passes=False)


def _zero_accum(accum, zbuf, sid):
  def zrow(i, c):
    zbuf[i, :] = jnp.zeros((F,), jnp.float32)
    return c
  lax.fori_loop(0, ZROWS, zrow, 0)
  for k in range(NPT // ZROWS):
    pltpu.sync_copy(zbuf, accum.at[pl.ds(sid * NPT + k * ZROWS, ZROWS)])


def _tile_rows(cid, sid):
  count = jnp.where(sid < RREM, RPT + 1, RPT)
  start = cid * RPC + sid * RPT + jnp.minimum(sid, RREM)
  return start, count


def _scatter_body(mode, *refs):
  if mode == "conv":
    edge, g, out, accum, zbuf, rbuf, sbuf, dbuf = refs
  else:
    edge, out, accum, zbuf, rbuf, dbuf = refs
  cid = lax.axis_index("c")
  sid = lax.axis_index("s")
  _zero_accum(accum, zbuf, sid)
  if mode == "deg":
    def orow(i, c):
      rbuf[i, :] = jnp.ones((F,), jnp.float32)
      return c
    lax.fori_loop(0, LANES, orow, 0)
  plsc.subcore_barrier()
  start, count = _tile_rows(cid, sid)

  def step(i, c):
    row = start + i
    pltpu.sync_copy(edge.at[1, pl.ds(row, 1)], dbuf)
    if mode == "conv":
      pltpu.sync_copy(edge.at[0, pl.ds(row, 1)], sbuf)
      pltpu.sync_copy(g.at[sbuf.at[0]], rbuf)
    pltpu.sync_copy(rbuf, accum.at[dbuf.at[0]], add=True)
    return c

  lax.fori_loop(0, count, step, 0)
  plsc.subcore_barrier()
  pltpu.sync_copy(accum.at[pl.ds(sid * NPT, NPT)], out.at[cid, sid])


def _make_scatter(mode):
  scratch = [
      pltpu.VMEM_SHARED((N, F), jnp.float32),
      pltpu.VMEM((ZROWS, F), jnp.float32),
      pltpu.VMEM((LANES, F), jnp.float32),
  ]
  if mode == "conv":
    scratch.append(pltpu.VMEM((1, LANES), jnp.int32))
  scratch.append(pltpu.VMEM((1, LANES), jnp.int32))
  return pl.kernel(
      functools.partial(_scatter_body, mode),
      out_type=jax.ShapeDtypeStruct((NC, NS, NPT, F), jnp.float32),
      mesh=_MESH(),
      scratch_types=scratch,
      compiler_params=_SC_PARAMS,
  )


def _decode_body(tab_hbm, eli, out, tab_v, i0, i1, ob):
  cid = lax.axis_index("c")
  sid = lax.axis_index("s")
  wid = sid * NC + cid
  pltpu.sync_copy(tab_hbm, tab_v)
  rpt = R // (NC * NS)
  rrem = R - rpt * NC * NS
  count = jnp.where(wid < rrem, rpt + 1, rpt)
  start = wid * rpt + jnp.minimum(wid, rrem)
  mask_hi = jnp.full((F,), -65536, jnp.int32)

  def step(i, c):
    row = start + i
    pltpu.sync_copy(eli.at[0, pl.ds(row, 1)], i0)
    pltpu.sync_copy(eli.at[1, pl.ds(row, 1)], i1)
    for k in range(LANES // F):
      ia = i0[0, pl.ds(k * F, F)]
      ib = i1[0, pl.ds(k * F, F)]
      va = plsc.load_gather(tab_v, [ia])
      vb = plsc.load_gather(tab_v, [ib])
      fa = plsc.bitcast(va & mask_hi, jnp.float32)
      fb = plsc.bitcast(lax.shift_left(vb, 16), jnp.float32)
      ob[0, pl.ds(k * F, F)] = fa + fb
    pltpu.sync_copy(ob, out.at[pl.ds(row, 1)])
    return c

  lax.fori_loop(0, count, step, 0)


_decode = pl.kernel(
    _decode_body,
    out_type=jax.ShapeDtypeStruct((R, LANES), jnp.float32),
    mesh=_MESH(),
    scratch_types=[
        pltpu.VMEM((N,), jnp.int32),
        pltpu.VMEM((1, LANES), jnp.int32),
        pltpu.VMEM((1, LANES), jnp.int32),
        pltpu.VMEM((1, LANES), jnp.float32),
    ],
    compiler_params=_SC_PARAMS,
)


NR = N // 8
LN = 128


def _enc_body(z_ref, w1_ref, degp_ref, g1_ref, dinv_ref):
  deg = degp_ref[0] + degp_ref[1] + 1.0
  dinv = lax.rsqrt(deg)
  dinv_ref[...] = dinv
  h = jnp.dot(z_ref[...], w1_ref[...], preferred_element_type=jnp.float32)
  g1_ref[...] = h * dinv


def _mid_body(s1p_ref, g1_ref, dinv_ref, b1_ref, w2_ref, g2_ref):
  dinv = dinv_ref[...]
  z1 = jax.nn.relu(dinv * (s1p_ref[0] + s1p_ref[1] + g1_ref[...])
                   + b1_ref[...])
  h2 = jnp.dot(z1, w2_ref[...], preferred_element_type=jnp.float32)
  g2_ref[...] = h2 * dinv


def _pack_body(s2p_ref, g2_ref, dinv_ref, b2_ref, sa_ref, sb_ref, fcb_ref,
               tab_ref):
  dinv = dinv_ref[...]
  z2 = dinv * (s2p_ref[0] + s2p_ref[1] + g2_ref[...]) + b2_ref[...]
  c = fcb_ref[0, 0] * 0.5
  y1 = jnp.dot(z2, sa_ref[...], preferred_element_type=jnp.float32) + c
  y2 = jnp.dot(z2, sb_ref[...], preferred_element_type=jnp.float32) + c
  ba = lax.bitcast_convert_type(y1, jnp.int32) + 0x8000
  bb = lax.bitcast_convert_type(y2, jnp.int32) + 0x8000
  hi = ba & jnp.int32(-65536)
  lo = lax.shift_right_logical(bb, 16) & 0xFFFF
  tab_ref[...] = hi | lo


def _full(shape):
  return pl.BlockSpec(shape, lambda: tuple(0 for _ in shape))


_enc = pl.pallas_call(
    _enc_body,
    in_specs=[_full((NR, LN)), _full((LN, LN)), _full((NC, NR, LN))],
    out_specs=[_full((NR, LN)), _full((NR, LN))],
    out_shape=[jax.ShapeDtypeStruct((NR, LN), jnp.float32),
               jax.ShapeDtypeStruct((NR, LN), jnp.float32)],
)

_mid = pl.pallas_call(
    _mid_body,
    in_specs=[_full((NC, NR, LN)), _full((NR, LN)), _full((NR, LN)),
              _full((1, LN)), _full((LN, LN))],
    out_specs=[_full((NR, LN))],
    out_shape=[jax.ShapeDtypeStruct((NR, LN), jnp.float32)],
)

_pack = pl.pallas_call(
    _pack_body,
    in_specs=[_full((NC, NR, LN)), _full((NR, LN)), _full((NR, LN)),
              _full((1, LN)), _full((LN, 8)), _full((LN, 8)), _full((1, 1))],
    out_specs=[_full((NR, 8))],
    out_shape=[jax.ShapeDtypeStruct((NR, 8), jnp.int32)],
)

_deg_scatter = _make_scatter("deg")
_conv_scatter = _make_scatter("conv")


def kernel(x, edge_index, edge_label_index, embed, W1, b1, W2, b2, fc_w, fc_b):
  z = jnp.take(embed, x, axis=0).reshape(NR, LN)
  ei = edge_index.reshape(2, R, LANES)
  eli = edge_label_index.reshape(2, R, LANES)
  eye8 = jnp.eye(8, dtype=jnp.float32)
  w1t = jnp.kron(eye8, W1)
  w2t = jnp.kron(eye8, W2)
  sa = jnp.kron(eye8, fc_w[:F, 0].reshape(F, 1))
  sb = jnp.kron(eye8, fc_w[F:, 0].reshape(F, 1))
  b1t = jnp.tile(b1, 8).reshape(1, LN)
  b2t = jnp.tile(b2, 8).reshape(1, LN)

  degp = _deg_scatter(ei).reshape(NC, NR, LN)
  g1, dinv = _enc(z, w1t, degp)
  s1p = _conv_scatter(ei, g1.reshape(N, F)).reshape(NC, NR, LN)
  (g2,) = _mid(s1p, g1, dinv, b1t, w2t)
  s2p = _conv_scatter(ei, g2.reshape(N, F)).reshape(NC, NR, LN)
  (tab,) = _pack(s2p, g2, dinv, b2t, sa, sb, fc_b.reshape(1, 1))
  dec = _decode(tab.reshape(N), eli)
  return dec.reshape(E, 1)


# speedup vs baseline: 86.1319x; 3.4976x over previous
import functools

import jax
import jax.numpy as jnp
from jax import lax
from jax.experimental import pallas as pl
from jax.experimental.pallas import tpu as pltpu
from jax.experimental.pallas import tpu_sc as plsc

N = 100000
E = 3200000
F = 16
LANES = 128
R = E // LANES
NC = 2
NS = 16
RPC = R // NC
RPT = RPC // NS
RREM = RPC - RPT * NS
NPT = N // NS
ZROWS = 625
NB = 20000
GRID = N // NB

_MESH = functools.partial(
    plsc.VectorSubcoreMesh, core_axis_name="c", subcore_axis_name="s",
    num_cores=NC, num_subcores=NS)
_SC_PARAMS = pltpu.CompilerParams(use_tc_tiling_on_sc=False,
                                  needs_layout_passes=False)


def _zero_accum(accum, zbuf, sid):
  def zrow(i, c):
    zbuf[i, :] = jnp.zeros((F,), jnp.float32)
    return c
  lax.fori_loop(0, ZROWS, zrow, 0)
  for k in range(NPT // ZROWS):
    pltpu.sync_copy(zbuf, accum.at[pl.ds(sid * NPT + k * ZROWS, ZROWS)])


U = 8
NU = R // U
UPT = NU // (NC * NS)
UREM = NU - UPT * NC * NS


def _unit_range(wid):
  count = jnp.where(wid < UREM, UPT + 1, UPT)
  start = wid * UPT + jnp.minimum(wid, UREM)
  return start, count


def _scatter_body(mode, *refs):
  if mode == "conv":
    edge, g, out, accum, zbuf, rbuf, sbuf, dbuf, sem_g, sem_s = refs
  else:
    edge, out, accum, zbuf, rbuf, dbuf, sem_s = refs
  cid = lax.axis_index("c")
  sid = lax.axis_index("s")
  wid = cid * NS + sid
  _zero_accum(accum, zbuf, sid)
  if mode == "deg":
    def orow(i, c):
      rbuf[i, :] = jnp.ones((F,), jnp.float32)
      return c
    lax.fori_loop(0, LANES, orow, 0)
  plsc.subcore_barrier()
  start, count = _unit_range(wid)

  def step(u, c):
    row0 = (start + u) * U
    pltpu.sync_copy(edge.at[1, pl.ds(row0, U)], dbuf)
    if mode == "conv":
      pltpu.sync_copy(edge.at[0, pl.ds(row0, U)], sbuf)
      gathers = [pltpu.async_copy(g.at[sbuf.at[j]], rbuf.at[j], sem_g)
                 for j in range(U)]
      for d in gathers:
        d.wait()
      scatters = [
          pltpu.async_copy(rbuf.at[j], accum.at[dbuf.at[j]], sem_s, add=True)
          for j in range(U)]
    else:
      scatters = [
          pltpu.async_copy(rbuf, accum.at[dbuf.at[j]], sem_s, add=True)
          for j in range(U)]
    for d in scatters:
      d.wait()
    return c

  lax.fori_loop(0, count, step, 0)
  plsc.subcore_barrier()
  pltpu.sync_copy(accum.at[pl.ds(sid * NPT, NPT)], out.at[cid, sid])


def _make_scatter(mode):
  scratch = [
      pltpu.VMEM_SHARED((N, F), jnp.float32),
      pltpu.VMEM((ZROWS, F), jnp.float32),
  ]
  if mode == "conv":
    scratch += [
        pltpu.VMEM((U, LANES, F), jnp.float32),
        pltpu.VMEM((U, LANES), jnp.int32),
        pltpu.VMEM((U, LANES), jnp.int32),
        pltpu.SemaphoreType.DMA,
        pltpu.SemaphoreType.DMA,
    ]
  else:
    scratch += [
        pltpu.VMEM((LANES, F), jnp.float32),
        pltpu.VMEM((U, LANES), jnp.int32),
        pltpu.SemaphoreType.DMA,
    ]
  return pl.kernel(
      functools.partial(_scatter_body, mode),
      out_type=jax.ShapeDtypeStruct((NC, NS, NPT, F), jnp.float32),
      mesh=_MESH(),
      scratch_types=scratch,
      compiler_params=_SC_PARAMS,
  )


def _decode_body(tab_hbm, eli, out, tab_v, i0, i1, ob):
  cid = lax.axis_index("c")
  sid = lax.axis_index("s")
  wid = cid * NS + sid
  pltpu.sync_copy(tab_hbm, tab_v)
  start, count = _unit_range(wid)
  mask_hi = jnp.full((F,), -65536, jnp.int32)

  def step(u, c):
    row0 = (start + u) * U
    pltpu.sync_copy(eli.at[0, pl.ds(row0, U)], i0)
    pltpu.sync_copy(eli.at[1, pl.ds(row0, U)], i1)
    for j in range(U):
      for k in range(LANES // F):
        ia = i0[j, pl.ds(k * F, F)]
        ib = i1[j, pl.ds(k * F, F)]
        va = plsc.load_gather(tab_v, [ia])
        vb = plsc.load_gather(tab_v, [ib])
        fa = plsc.bitcast(va & mask_hi, jnp.float32)
        fb = plsc.bitcast(lax.shift_left(vb, 16), jnp.float32)
        ob[j, pl.ds(k * F, F)] = fa + fb
    pltpu.sync_copy(ob, out.at[pl.ds(row0, U)])
    return c

  lax.fori_loop(0, count, step, 0)


_decode = pl.kernel(
    _decode_body,
    out_type=jax.ShapeDtypeStruct((R, LANES), jnp.float32),
    mesh=_MESH(),
    scratch_types=[
        pltpu.VMEM((N,), jnp.int32),
        pltpu.VMEM((U, LANES), jnp.int32),
        pltpu.VMEM((U, LANES), jnp.int32),
        pltpu.VMEM((U, LANES), jnp.float32),
    ],
    compiler_params=_SC_PARAMS,
)


NR = N // 8
LN = 128


def _enc_body(z_ref, w1_ref, degp_ref, g1_ref, dinv_ref):
  deg = degp_ref[0] + degp_ref[1] + 1.0
  dinv = lax.rsqrt(deg)
  dinv_ref[...] = dinv
  h = jnp.dot(z_ref[...], w1_ref[...], preferred_element_type=jnp.float32)
  g1_ref[...] = h * dinv


def _mid_body(s1p_ref, g1_ref, dinv_ref, b1_ref, w2_ref, g2_ref):
  dinv = dinv_ref[...]
  z1 = jax.nn.relu(dinv * (s1p_ref[0] + s1p_ref[1] + g1_ref[...])
                   + b1_ref[...])
  h2 = jnp.dot(z1, w2_ref[...], preferred_element_type=jnp.float32)
  g2_ref[...] = h2 * dinv


def _pack_body(s2p_ref, g2_ref, dinv_ref, b2_ref, sa_ref, sb_ref, fcb_ref,
               tab_ref):
  dinv = dinv_ref[...]
  z2 = dinv * (s2p_ref[0] + s2p_ref[1] + g2_ref[...]) + b2_ref[...]
  c = fcb_ref[0, 0] * 0.5
  y1 = jnp.dot(z2, sa_ref[...], preferred_element_type=jnp.float32) + c
  y2 = jnp.dot(z2, sb_ref[...], preferred_element_type=jnp.float32) + c
  ba = lax.bitcast_convert_type(y1, jnp.int32) + 0x8000
  bb = lax.bitcast_convert_type(y2, jnp.int32) + 0x8000
  hi = ba & jnp.int32(-65536)
  lo = lax.shift_right_logical(bb, 16) & 0xFFFF
  tab_ref[...] = hi | lo


def _full(shape):
  return pl.BlockSpec(shape, lambda: tuple(0 for _ in shape))


_enc = pl.pallas_call(
    _enc_body,
    in_specs=[_full((NR, LN)), _full((LN, LN)), _full((NC, NR, LN))],
    out_specs=[_full((NR, LN)), _full((NR, LN))],
    out_shape=[jax.ShapeDtypeStruct((NR, LN), jnp.float32),
               jax.ShapeDtypeStruct((NR, LN), jnp.float32)],
)

_mid = pl.pallas_call(
    _mid_body,
    in_specs=[_full((NC, NR, LN)), _full((NR, LN)), _full((NR, LN)),
              _full((1, LN)), _full((LN, LN))],
    out_specs=[_full((NR, LN))],
    out_shape=[jax.ShapeDtypeStruct((NR, LN), jnp.float32)],
)

_pack = pl.pallas_call(
    _pack_body,
    in_specs=[_full((NC, NR, LN)), _full((NR, LN)), _full((NR, LN)),
              _full((1, LN)), _full((LN, 8)), _full((LN, 8)), _full((1, 1))],
    out_specs=[_full((NR, 8))],
    out_shape=[jax.ShapeDtypeStruct((NR, 8), jnp.int32)],
)

_deg_scatter = _make_scatter("deg")
_conv_scatter = _make_scatter("conv")


def kernel(x, edge_index, edge_label_index, embed, W1, b1, W2, b2, fc_w, fc_b):
  z = jnp.take(embed, x, axis=0).reshape(NR, LN)
  ei = edge_index.reshape(2, R, LANES)
  eli = edge_label_index.reshape(2, R, LANES)
  eye8 = jnp.eye(8, dtype=jnp.float32)
  w1t = jnp.kron(eye8, W1)
  w2t = jnp.kron(eye8, W2)
  sa = jnp.kron(eye8, fc_w[:F, 0].reshape(F, 1))
  sb = jnp.kron(eye8, fc_w[F:, 0].reshape(F, 1))
  b1t = jnp.tile(b1, 8).reshape(1, LN)
  b2t = jnp.tile(b2, 8).reshape(1, LN)

  degp = _deg_scatter(ei).reshape(NC, NR, LN)
  g1, dinv = _enc(z, w1t, degp)
  s1p = _conv_scatter(ei, g1.reshape(N, F)).reshape(NC, NR, LN)
  (g2,) = _mid(s1p, g1, dinv, b1t, w2t)
  s2p = _conv_scatter(ei, g2.reshape(N, F)).reshape(NC, NR, LN)
  (tab,) = _pack(s2p, g2, dinv, b2t, sa, sb, fc_b.reshape(1, 1))
  dec = _decode(tab.reshape(N), eli)
  return dec.reshape(E, 1)
